# baseline (speedup 1.0000x reference)
import functools

import jax
import jax.numpy as jnp
from jax.experimental import pallas as pl
from jax.experimental.pallas import tpu as pltpu
from jax.experimental.pallas import tpu_sc as plsc


def _prep_kernel(fm_a_ref, fm_q_ref, corr_ref,
                 ta_ref, tq_ref, ta16_ref, tq16_ref, ia_ref, iq_ref,
                 *, D, FH, FW):
    b = pl.program_id(0)
    HW = FH * FW
    for fm_ref, t_ref, t16_ref in ((fm_a_ref, ta_ref, ta16_ref),
                                   (fm_q_ref, tq_ref, tq16_ref)):
        f = fm_ref[0]
        n = jnp.sqrt(jnp.sum(f * f, axis=0, keepdims=True))
        fmn = f / jnp.maximum(n, 1e-8)
        fmnt = fmn.T
        t_ref[0] = fmnt
        t16_ref[0] = fmnt.astype(jnp.bfloat16)
    c = corr_ref[0]
    g = jnp.clip(c // 8, 0, FH - 1)
    base = b * HW
    ia_ref[0] = base + g[:, 0:1] * FW + g[:, 1:2]
    iq_ref[0] = base + g[:, 2:3] * FW + g[:, 3:4]


def _sc_gather_call(ta, tq, ia, iq, n_rows, d):
    info = plsc.get_sparse_core_info()
    nc, ns = info.num_cores, info.num_subcores
    nw = nc * ns
    rows_per_w = n_rows // nw
    chunk = min(128, rows_per_w)
    n_chunks = rows_per_w // chunk
    mesh = plsc.VectorSubcoreMesh(core_axis_name="c", subcore_axis_name="s")

    @functools.partial(
        pl.kernel, mesh=mesh,
        out_type=[jax.ShapeDtypeStruct((n_rows, d), jnp.float32),
                  jax.ShapeDtypeStruct((n_rows, d), jnp.float32)],
        scratch_types=[pltpu.VMEM((rows_per_w,), jnp.int32),
                       pltpu.VMEM((rows_per_w,), jnp.int32),
                       pltpu.VMEM((chunk, d), jnp.float32),
                       pltpu.VMEM((chunk, d), jnp.float32),
                       pltpu.SemaphoreType.DMA,
                       pltpu.SemaphoreType.DMA,
                       pltpu.SemaphoreType.DMA,
                       pltpu.SemaphoreType.DMA],
    )
    def k(ta_hbm, tq_hbm, ia_hbm, iq_hbm, oa, oq,
          idx_a, idx_q, r0, r1, s0, s1, w0, w1):
        wid = jax.lax.axis_index("s") * nc + jax.lax.axis_index("c")
        base = wid * rows_per_w
        pltpu.sync_copy(ia_hbm.at[pl.ds(base, rows_per_w)], idx_a)
        pltpu.sync_copy(iq_hbm.at[pl.ds(base, rows_per_w)], idx_q)
        for t_hbm, i_v, o_hbm in ((ta_hbm, idx_a, oa), (tq_hbm, idx_q, oq)):
            for j in range(0, n_chunks, 2):
                g0 = pltpu.async_copy(
                    t_hbm.at[i_v.at[pl.ds(j * chunk, chunk)]], r0, s0)
                g1 = None
                if j + 1 < n_chunks:
                    g1 = pltpu.async_copy(
                        t_hbm.at[i_v.at[pl.ds((j + 1) * chunk, chunk)]],
                        r1, s1)
                g0.wait()
                c0 = pltpu.async_copy(
                    r0, o_hbm.at[pl.ds(base + j * chunk, chunk)], w0)
                if g1 is not None:
                    g1.wait()
                    c1 = pltpu.async_copy(
                        r1, o_hbm.at[pl.ds(base + (j + 1) * chunk, chunk)],
                        w1)
                    c1.wait()
                c0.wait()

    return k(ta, tq, ia, iq)

POS_MARGIN = 0.1
NEG_MARGIN = 1.4
NEG_KERNEL = 9.0
MASK_TH = 0.5


def _main_kernel(t_a_ref, t_q_ref, pa_ref, pq_ref, corr_ref, valid_ref,
                 idxa_ref, idxq_ref, pos_ref, nega_ref, negq_ref,
                 *, R, D, FH, FW):
    HW = FH * FW
    b = pl.program_id(0)
    t = pl.program_id(1)

    @pl.when(jnp.logical_and(b == 0, t == 0))
    def _():
        pos_ref[...] = jnp.zeros_like(pos_ref)
        nega_ref[...] = jnp.zeros_like(nega_ref)
        negq_ref[...] = jnp.zeros_like(negq_ref)

    vm = (valid_ref[0, 0, 0] == 1).astype(jnp.float32)
    corr = corr_ref[0, 0]
    gt = jnp.clip(corr // 8, 0, FH - 1)

    col = jax.lax.broadcasted_iota(jnp.int32, (R, HW), 1)
    yy16 = (col // FW).astype(jnp.bfloat16)
    xx16 = (col % FW).astype(jnp.bfloat16)

    def one_side(t_ref, p_ref, y, x):
        p16 = (p_ref[0, 0] * vm).astype(jnp.bfloat16)
        mh = jax.lax.dot_general(
            p16 * jnp.bfloat16(-0.5), t_ref[0],
            (((1,), (1,)), ((), ())),
            precision=jax.lax.Precision.DEFAULT,
            preferred_element_type=jnp.float32)
        fd = 0.5 + mh
        y16 = y.astype(jnp.bfloat16)
        x16 = x.astype(jnp.bfloat16)
        dy = y16 - yy16
        dx = x16 - xx16
        d2 = dy * dy + dx * dx
        fdp = jnp.where(d2 < jnp.bfloat16(81.0), 1e9, fd)
        minv = jnp.min(fdp, axis=1, keepdims=True)
        idx = jnp.argmin(fdp, axis=1).astype(jnp.int32)[:, None]
        dist_neg = minv
        idx_f = idx.astype(jnp.float32)
        ny = jnp.floor(idx_f * jnp.float32(1.0 / FW))
        nx = idx_f - ny * jnp.float32(FW)
        posn = p16.astype(jnp.float32)
        return posn, dist_neg, jnp.concatenate([ny * vm, nx * vm], axis=1)

    posn_a, dneg_a, oidx_a = one_side(t_a_ref, pa_ref, gt[:, 0:1], gt[:, 1:2])
    posn_q, dneg_q, oidx_q = one_side(t_q_ref, pq_ref, gt[:, 2:3], gt[:, 3:4])
    idxa_ref[0, 0] = oidx_a
    idxq_ref[0, 0] = oidx_q

    dist_pos = 0.5 * (1.0 - jnp.sum(posn_a * posn_q, axis=1, keepdims=True))
    pos_ref[...] = pos_ref[...] + vm * jnp.sum(jax.nn.relu(dist_pos - POS_MARGIN))
    nega_ref[...] = nega_ref[...] + vm * jnp.sum(jax.nn.relu(NEG_MARGIN - dneg_a))
    negq_ref[...] = negq_ref[...] + vm * jnp.sum(jax.nn.relu(NEG_MARGIN - dneg_q))


def _mask_kernel(la_ref, lq_ref, ma_ref, mq_ref,
                 pma_ref, pmq_ref, bcea_ref, bceq_ref, ioua_ref, iouq_ref,
                 *, B, FH, FW):
    b = pl.program_id(0)

    @pl.when(b == 0)
    def _():
        bcea_ref[...] = jnp.zeros_like(bcea_ref)
        bceq_ref[...] = jnp.zeros_like(bceq_ref)
        ioua_ref[...] = jnp.zeros_like(ioua_ref)
        iouq_ref[...] = jnp.zeros_like(iouq_ref)

    last = b == B - 1

    def one(l_ref, m_ref, pm_ref, bce_ref, iou_ref):
        x = l_ref[0]
        z = m_ref[0].astype(jnp.float32)
        bce = jnp.sum(jax.nn.relu(x) - x * z + jnp.log1p(jnp.exp(-jnp.abs(x))))
        tot = bce_ref[...] + bce
        bce_ref[...] = jnp.where(last, tot / (B * FH * FW), tot)
        pred = (jax.nn.sigmoid(x) > MASK_TH).astype(jnp.int32)
        pm_ref[0] = pred
        pf = pred.astype(jnp.float32)
        inter = jnp.sum(z * pf)
        union = jnp.sum(jnp.clip(z + pf, 0.0, 1.0))
        itot = iou_ref[...] + inter / (union + 1e-6)
        iou_ref[...] = jnp.where(last, itot / B, itot)

    one(la_ref, ma_ref, pma_ref, bcea_ref, ioua_ref)
    one(lq_ref, mq_ref, pmq_ref, bceq_ref, iouq_ref)


def kernel(featmap_a, featmap_q, mask_a_logits, mask_q_logits, corrs, valid,
           anchor_rgb, anchor_mask, query_mask):
    B, D, FH, FW = featmap_a.shape
    HW = FH * FW
    N = corrs.shape[1]
    R = 1024
    NT = N // R

    fa = featmap_a.reshape(B, D, HW)
    fq = featmap_q.reshape(B, D, HW)
    corr4 = corrs.reshape(B, NT, R, 4)
    valid3 = valid.reshape(B, 1, 1)

    prep = pl.pallas_call(
        functools.partial(_prep_kernel, D=D, FH=FH, FW=FW),
        grid=(B,),
        in_specs=[
            pl.BlockSpec((1, D, HW), lambda b: (b, 0, 0)),
            pl.BlockSpec((1, D, HW), lambda b: (b, 0, 0)),
            pl.BlockSpec((1, N, 4), lambda b: (b, 0, 0)),
        ],
        out_specs=[
            pl.BlockSpec((1, HW, D), lambda b: (b, 0, 0)),
            pl.BlockSpec((1, HW, D), lambda b: (b, 0, 0)),
            pl.BlockSpec((1, HW, D), lambda b: (b, 0, 0)),
            pl.BlockSpec((1, HW, D), lambda b: (b, 0, 0)),
            pl.BlockSpec((1, N, 1), lambda b: (b, 0, 0)),
            pl.BlockSpec((1, N, 1), lambda b: (b, 0, 0)),
        ],
        out_shape=[
            jax.ShapeDtypeStruct((B, HW, D), jnp.float32),
            jax.ShapeDtypeStruct((B, HW, D), jnp.float32),
            jax.ShapeDtypeStruct((B, HW, D), jnp.bfloat16),
            jax.ShapeDtypeStruct((B, HW, D), jnp.bfloat16),
            jax.ShapeDtypeStruct((B, N, 1), jnp.int32),
            jax.ShapeDtypeStruct((B, N, 1), jnp.int32),
        ],
    )
    ta, tq, ta16, tq16, ia, iq = prep(fa, fq, corrs)

    oa, oq = _sc_gather_call(ta.reshape(B * HW, D), tq.reshape(B * HW, D),
                             ia.reshape(B * N), iq.reshape(B * N), B * N, D)
    pa = oa.reshape(B, NT, R, D)
    pq = oq.reshape(B, NT, R, D)

    main = pl.pallas_call(
        functools.partial(_main_kernel, R=R, D=D, FH=FH, FW=FW),
        grid=(B, NT),
        in_specs=[
            pl.BlockSpec((1, HW, D), lambda b, t: (b, 0, 0)),
            pl.BlockSpec((1, HW, D), lambda b, t: (b, 0, 0)),
            pl.BlockSpec((1, 1, R, D), lambda b, t: (b, t, 0, 0)),
            pl.BlockSpec((1, 1, R, D), lambda b, t: (b, t, 0, 0)),
            pl.BlockSpec((1, 1, R, 4), lambda b, t: (b, t, 0, 0)),
            pl.BlockSpec((1, 1, 1), lambda b, t: (b, 0, 0)),
        ],
        out_specs=[
            pl.BlockSpec((1, 1, R, 2), lambda b, t: (b, t, 0, 0)),
            pl.BlockSpec((1, 1, R, 2), lambda b, t: (b, t, 0, 0)),
            pl.BlockSpec((1, 1), lambda b, t: (0, 0)),
            pl.BlockSpec((1, 1), lambda b, t: (0, 0)),
            pl.BlockSpec((1, 1), lambda b, t: (0, 0)),
        ],
        out_shape=[
            jax.ShapeDtypeStruct((B, NT, R, 2), jnp.float32),
            jax.ShapeDtypeStruct((B, NT, R, 2), jnp.float32),
            jax.ShapeDtypeStruct((1, 1), jnp.float32),
            jax.ShapeDtypeStruct((1, 1), jnp.float32),
            jax.ShapeDtypeStruct((1, 1), jnp.float32),
        ],
    )
    idxa, idxq, pos_s, nega_s, negq_s = main(ta16, tq16, pa, pq,
                                             corr4, valid3)

    ma = anchor_mask[:, ::8, ::8]
    mq = query_mask[:, ::8, ::8]
    la = mask_a_logits.reshape(B, FH, FW)
    lq = mask_q_logits.reshape(B, FH, FW)
    mask_out = pl.pallas_call(
        functools.partial(_mask_kernel, B=B, FH=FH, FW=FW),
        grid=(B,),
        in_specs=[
            pl.BlockSpec((1, FH, FW), lambda b: (b, 0, 0)),
            pl.BlockSpec((1, FH, FW), lambda b: (b, 0, 0)),
            pl.BlockSpec((1, FH, FW), lambda b: (b, 0, 0)),
            pl.BlockSpec((1, FH, FW), lambda b: (b, 0, 0)),
        ],
        out_specs=[
            pl.BlockSpec((1, FH, FW), lambda b: (b, 0, 0)),
            pl.BlockSpec((1, FH, FW), lambda b: (b, 0, 0)),
            pl.BlockSpec((1, 1), lambda b: (0, 0)),
            pl.BlockSpec((1, 1), lambda b: (0, 0)),
            pl.BlockSpec((1, 1), lambda b: (0, 0)),
            pl.BlockSpec((1, 1), lambda b: (0, 0)),
        ],
        out_shape=[
            jax.ShapeDtypeStruct((B, FH, FW), jnp.int32),
            jax.ShapeDtypeStruct((B, FH, FW), jnp.int32),
            jax.ShapeDtypeStruct((1, 1), jnp.float32),
            jax.ShapeDtypeStruct((1, 1), jnp.float32),
            jax.ShapeDtypeStruct((1, 1), jnp.float32),
            jax.ShapeDtypeStruct((1, 1), jnp.float32),
        ],
    )
    pm_a, pm_q, bce_a, bce_q, iou_a, iou_q = mask_out(la, lq, ma, mq)

    vmask = (valid == 1).astype(jnp.float32)
    cnt = jnp.sum(vmask)
    denom = jnp.maximum(cnt, 1.0)
    pos_loss = jnp.where(cnt > 0, (pos_s[0, 0] / N) / denom, 0.0)
    neg_loss_a = jnp.where(cnt > 0, (nega_s[0, 0] / N) / denom, 0.0)
    neg_loss_q = jnp.where(cnt > 0, (negq_s[0, 0] / N) / denom, 0.0)
    losses = jnp.stack([0.5 * (bce_a[0, 0] + bce_q[0, 0]), pos_loss,
                        0.5 * (neg_loss_a + neg_loss_q)])
    return (losses,
            idxa.reshape(B, N, 2), idxq.reshape(B, N, 2),
            pm_a, pm_q,
            iou_a[0, 0], iou_q[0, 0])


# speedup vs baseline: 1.0041x; 1.0041x over previous
import functools

import jax
import jax.numpy as jnp
from jax.experimental import pallas as pl
from jax.experimental.pallas import tpu as pltpu
from jax.experimental.pallas import tpu_sc as plsc


def _prep_kernel(fm_a_ref, fm_q_ref, corr_ref,
                 ta_ref, tq_ref, ta16_ref, tq16_ref, ia_ref, iq_ref,
                 *, D, FH, FW):
    b = pl.program_id(0)
    HW = FH * FW
    for fm_ref, t_ref, t16_ref in ((fm_a_ref, ta_ref, ta16_ref),
                                   (fm_q_ref, tq_ref, tq16_ref)):
        f = fm_ref[0]
        n = jnp.sqrt(jnp.sum(f * f, axis=0, keepdims=True))
        fmn = f / jnp.maximum(n, 1e-8)
        fmnt = fmn.T
        t_ref[0] = fmnt
        t16_ref[0] = fmnt.astype(jnp.bfloat16)
    c = corr_ref[0]
    g = jnp.clip(c // 8, 0, FH - 1)
    base = b * HW
    ia_ref[0] = base + g[:, 0:1] * FW + g[:, 1:2]
    iq_ref[0] = base + g[:, 2:3] * FW + g[:, 3:4]


def _sc_gather_call(ta, tq, ia, iq, n_rows, d):
    info = plsc.get_sparse_core_info()
    nc, ns = info.num_cores, info.num_subcores
    nw = nc * ns
    rows_per_w = n_rows // nw
    chunk = min(64, rows_per_w)
    n_chunks = rows_per_w // chunk
    mesh = plsc.VectorSubcoreMesh(core_axis_name="c", subcore_axis_name="s")

    @functools.partial(
        pl.kernel, mesh=mesh,
        out_type=[jax.ShapeDtypeStruct((n_rows, d), jnp.float32),
                  jax.ShapeDtypeStruct((n_rows, d), jnp.float32)],
        scratch_types=[pltpu.VMEM((chunk,), jnp.int32),
                       pltpu.VMEM((chunk, d), jnp.float32),
                       pltpu.SemaphoreType.DMA],
    )
    def k(ta_hbm, tq_hbm, ia_hbm, iq_hbm, oa, oq, idx_v, rows_v, sem):
        wid = jax.lax.axis_index("s") * nc + jax.lax.axis_index("c")
        for t_hbm, i_hbm, o_hbm in ((ta_hbm, ia_hbm, oa),
                                    (tq_hbm, iq_hbm, oq)):
            for j in range(n_chunks):
                base = wid * rows_per_w + j * chunk
                pltpu.sync_copy(i_hbm.at[pl.ds(base, chunk)], idx_v)
                pltpu.async_copy(t_hbm.at[idx_v], rows_v, sem).wait()
                pltpu.sync_copy(rows_v, o_hbm.at[pl.ds(base, chunk)])

    return k(ta, tq, ia, iq)

POS_MARGIN = 0.1
NEG_MARGIN = 1.4
NEG_KERNEL = 9.0
MASK_TH = 0.5


def _main_kernel(t_a_ref, t_q_ref, pa_ref, pq_ref, corr_ref, valid_ref,
                 idxa_ref, idxq_ref, pos_ref, nega_ref, negq_ref,
                 *, R, D, FH, FW):
    HW = FH * FW
    b = pl.program_id(0)
    t = pl.program_id(1)

    @pl.when(jnp.logical_and(b == 0, t == 0))
    def _():
        pos_ref[...] = jnp.zeros_like(pos_ref)
        nega_ref[...] = jnp.zeros_like(nega_ref)
        negq_ref[...] = jnp.zeros_like(negq_ref)

    vm = (valid_ref[0, 0, 0] == 1).astype(jnp.float32)
    corr = corr_ref[0, 0]
    gt = jnp.clip(corr // 8, 0, FH - 1)

    col = jax.lax.broadcasted_iota(jnp.int32, (R, HW), 1)
    yy16 = (col // FW).astype(jnp.bfloat16)
    xx16 = (col % FW).astype(jnp.bfloat16)

    def one_side(t_ref, p_ref, y, x):
        p16 = (p_ref[0, 0] * vm).astype(jnp.bfloat16)
        mh = jax.lax.dot_general(
            p16 * jnp.bfloat16(-0.5), t_ref[0],
            (((1,), (1,)), ((), ())),
            precision=jax.lax.Precision.DEFAULT,
            preferred_element_type=jnp.float32)
        fd = 0.5 + mh
        y16 = y.astype(jnp.bfloat16)
        x16 = x.astype(jnp.bfloat16)
        dy = y16 - yy16
        dx = x16 - xx16
        d2 = dy * dy + dx * dx
        fdp = jnp.where(d2 < jnp.bfloat16(81.0), 1e9, fd)
        minv = jnp.min(fdp, axis=1, keepdims=True)
        idx = jnp.argmin(fdp, axis=1).astype(jnp.int32)[:, None]
        dist_neg = minv
        idx_f = idx.astype(jnp.float32)
        ny = jnp.floor(idx_f * jnp.float32(1.0 / FW))
        nx = idx_f - ny * jnp.float32(FW)
        posn = p16.astype(jnp.float32)
        return posn, dist_neg, jnp.concatenate([ny * vm, nx * vm], axis=1)

    posn_a, dneg_a, oidx_a = one_side(t_a_ref, pa_ref, gt[:, 0:1], gt[:, 1:2])
    posn_q, dneg_q, oidx_q = one_side(t_q_ref, pq_ref, gt[:, 2:3], gt[:, 3:4])
    idxa_ref[0, 0] = oidx_a
    idxq_ref[0, 0] = oidx_q

    dist_pos = 0.5 * (1.0 - jnp.sum(posn_a * posn_q, axis=1, keepdims=True))
    pos_ref[...] = pos_ref[...] + vm * jnp.sum(jax.nn.relu(dist_pos - POS_MARGIN))
    nega_ref[...] = nega_ref[...] + vm * jnp.sum(jax.nn.relu(NEG_MARGIN - dneg_a))
    negq_ref[...] = negq_ref[...] + vm * jnp.sum(jax.nn.relu(NEG_MARGIN - dneg_q))


def _mask_kernel(la_ref, lq_ref, ma_ref, mq_ref,
                 pma_ref, pmq_ref, bcea_ref, bceq_ref, ioua_ref, iouq_ref,
                 *, B, FH, FW):
    b = pl.program_id(0)

    @pl.when(b == 0)
    def _():
        bcea_ref[...] = jnp.zeros_like(bcea_ref)
        bceq_ref[...] = jnp.zeros_like(bceq_ref)
        ioua_ref[...] = jnp.zeros_like(ioua_ref)
        iouq_ref[...] = jnp.zeros_like(iouq_ref)

    last = b == B - 1

    def one(l_ref, m_ref, pm_ref, bce_ref, iou_ref):
        x = l_ref[0]
        z = m_ref[0].astype(jnp.float32)
        bce = jnp.sum(jax.nn.relu(x) - x * z + jnp.log1p(jnp.exp(-jnp.abs(x))))
        tot = bce_ref[...] + bce
        bce_ref[...] = jnp.where(last, tot / (B * FH * FW), tot)
        pred = (jax.nn.sigmoid(x) > MASK_TH).astype(jnp.int32)
        pm_ref[0] = pred
        pf = pred.astype(jnp.float32)
        inter = jnp.sum(z * pf)
        union = jnp.sum(jnp.clip(z + pf, 0.0, 1.0))
        itot = iou_ref[...] + inter / (union + 1e-6)
        iou_ref[...] = jnp.where(last, itot / B, itot)

    one(la_ref, ma_ref, pma_ref, bcea_ref, ioua_ref)
    one(lq_ref, mq_ref, pmq_ref, bceq_ref, iouq_ref)


def kernel(featmap_a, featmap_q, mask_a_logits, mask_q_logits, corrs, valid,
           anchor_rgb, anchor_mask, query_mask):
    B, D, FH, FW = featmap_a.shape
    HW = FH * FW
    N = corrs.shape[1]
    R = 1024
    NT = N // R

    fa = featmap_a.reshape(B, D, HW)
    fq = featmap_q.reshape(B, D, HW)
    corr4 = corrs.reshape(B, NT, R, 4)
    valid3 = valid.reshape(B, 1, 1)

    prep = pl.pallas_call(
        functools.partial(_prep_kernel, D=D, FH=FH, FW=FW),
        grid=(B,),
        in_specs=[
            pl.BlockSpec((1, D, HW), lambda b: (b, 0, 0)),
            pl.BlockSpec((1, D, HW), lambda b: (b, 0, 0)),
            pl.BlockSpec((1, N, 4), lambda b: (b, 0, 0)),
        ],
        out_specs=[
            pl.BlockSpec((1, HW, D), lambda b: (b, 0, 0)),
            pl.BlockSpec((1, HW, D), lambda b: (b, 0, 0)),
            pl.BlockSpec((1, HW, D), lambda b: (b, 0, 0)),
            pl.BlockSpec((1, HW, D), lambda b: (b, 0, 0)),
            pl.BlockSpec((1, N, 1), lambda b: (b, 0, 0)),
            pl.BlockSpec((1, N, 1), lambda b: (b, 0, 0)),
        ],
        out_shape=[
            jax.ShapeDtypeStruct((B, HW, D), jnp.float32),
            jax.ShapeDtypeStruct((B, HW, D), jnp.float32),
            jax.ShapeDtypeStruct((B, HW, D), jnp.bfloat16),
            jax.ShapeDtypeStruct((B, HW, D), jnp.bfloat16),
            jax.ShapeDtypeStruct((B, N, 1), jnp.int32),
            jax.ShapeDtypeStruct((B, N, 1), jnp.int32),
        ],
    )
    ta, tq, ta16, tq16, ia, iq = prep(fa, fq, corrs)

    oa, oq = _sc_gather_call(ta.reshape(B * HW, D), tq.reshape(B * HW, D),
                             ia.reshape(B * N), iq.reshape(B * N), B * N, D)
    pa = oa.reshape(B, NT, R, D)
    pq = oq.reshape(B, NT, R, D)

    main = pl.pallas_call(
        functools.partial(_main_kernel, R=R, D=D, FH=FH, FW=FW),
        grid=(B, NT),
        in_specs=[
            pl.BlockSpec((1, HW, D), lambda b, t: (b, 0, 0)),
            pl.BlockSpec((1, HW, D), lambda b, t: (b, 0, 0)),
            pl.BlockSpec((1, 1, R, D), lambda b, t: (b, t, 0, 0)),
            pl.BlockSpec((1, 1, R, D), lambda b, t: (b, t, 0, 0)),
            pl.BlockSpec((1, 1, R, 4), lambda b, t: (b, t, 0, 0)),
            pl.BlockSpec((1, 1, 1), lambda b, t: (b, 0, 0)),
        ],
        out_specs=[
            pl.BlockSpec((1, 1, R, 2), lambda b, t: (b, t, 0, 0)),
            pl.BlockSpec((1, 1, R, 2), lambda b, t: (b, t, 0, 0)),
            pl.BlockSpec((1, 1), lambda b, t: (0, 0)),
            pl.BlockSpec((1, 1), lambda b, t: (0, 0)),
            pl.BlockSpec((1, 1), lambda b, t: (0, 0)),
        ],
        out_shape=[
            jax.ShapeDtypeStruct((B, NT, R, 2), jnp.float32),
            jax.ShapeDtypeStruct((B, NT, R, 2), jnp.float32),
            jax.ShapeDtypeStruct((1, 1), jnp.float32),
            jax.ShapeDtypeStruct((1, 1), jnp.float32),
            jax.ShapeDtypeStruct((1, 1), jnp.float32),
        ],
    )
    idxa, idxq, pos_s, nega_s, negq_s = main(ta16, tq16, pa, pq,
                                             corr4, valid3)

    ma = anchor_mask[:, ::8, ::8]
    mq = query_mask[:, ::8, ::8]
    la = mask_a_logits.reshape(B, FH, FW)
    lq = mask_q_logits.reshape(B, FH, FW)
    mask_out = pl.pallas_call(
        functools.partial(_mask_kernel, B=B, FH=FH, FW=FW),
        grid=(B,),
        in_specs=[
            pl.BlockSpec((1, FH, FW), lambda b: (b, 0, 0)),
            pl.BlockSpec((1, FH, FW), lambda b: (b, 0, 0)),
            pl.BlockSpec((1, FH, FW), lambda b: (b, 0, 0)),
            pl.BlockSpec((1, FH, FW), lambda b: (b, 0, 0)),
        ],
        out_specs=[
            pl.BlockSpec((1, FH, FW), lambda b: (b, 0, 0)),
            pl.BlockSpec((1, FH, FW), lambda b: (b, 0, 0)),
            pl.BlockSpec((1, 1), lambda b: (0, 0)),
            pl.BlockSpec((1, 1), lambda b: (0, 0)),
            pl.BlockSpec((1, 1), lambda b: (0, 0)),
            pl.BlockSpec((1, 1), lambda b: (0, 0)),
        ],
        out_shape=[
            jax.ShapeDtypeStruct((B, FH, FW), jnp.int32),
            jax.ShapeDtypeStruct((B, FH, FW), jnp.int32),
            jax.ShapeDtypeStruct((1, 1), jnp.float32),
            jax.ShapeDtypeStruct((1, 1), jnp.float32),
            jax.ShapeDtypeStruct((1, 1), jnp.float32),
            jax.ShapeDtypeStruct((1, 1), jnp.float32),
        ],
    )
    pm_a, pm_q, bce_a, bce_q, iou_a, iou_q = mask_out(la, lq, ma, mq)

    vmask = (valid == 1).astype(jnp.float32)
    cnt = jnp.sum(vmask)
    denom = jnp.maximum(cnt, 1.0)
    pos_loss = jnp.where(cnt > 0, (pos_s[0, 0] / N) / denom, 0.0)
    neg_loss_a = jnp.where(cnt > 0, (nega_s[0, 0] / N) / denom, 0.0)
    neg_loss_q = jnp.where(cnt > 0, (negq_s[0, 0] / N) / denom, 0.0)
    losses = jnp.stack([0.5 * (bce_a[0, 0] + bce_q[0, 0]), pos_loss,
                        0.5 * (neg_loss_a + neg_loss_q)])
    return (losses,
            idxa.reshape(B, N, 2), idxq.reshape(B, N, 2),
            pm_a, pm_q,
            iou_a[0, 0], iou_q[0, 0])


# speedup vs baseline: 1.0145x; 1.0103x over previous
import functools

import jax
import jax.numpy as jnp
from jax.experimental import pallas as pl
from jax.experimental.pallas import tpu as pltpu
from jax.experimental.pallas import tpu_sc as plsc


def _prep_kernel(fm_a_ref, fm_q_ref, corr_ref,
                 ta_ref, tq_ref, ia_ref, iq_ref, *, D, FH, FW):
    b = pl.program_id(0)
    HW = FH * FW
    for fm_ref, t_ref in ((fm_a_ref, ta_ref), (fm_q_ref, tq_ref)):
        f = fm_ref[0]
        n = jnp.sqrt(jnp.sum(f * f, axis=0, keepdims=True))
        fmn = f / jnp.maximum(n, 1e-8)
        t_ref[0] = fmn.T
    c = corr_ref[0]
    g = jnp.clip(c // 8, 0, FH - 1)
    base = b * HW
    ia_ref[0] = base + g[:, 0:1] * FW + g[:, 1:2]
    iq_ref[0] = base + g[:, 2:3] * FW + g[:, 3:4]


def _sc_gather_call(ta, tq, ia, iq, n_rows, d):
    info = plsc.get_sparse_core_info()
    nc, ns = info.num_cores, info.num_subcores
    nw = nc * ns
    rows_per_w = n_rows // nw
    chunk = min(64, rows_per_w)
    n_chunks = rows_per_w // chunk
    mesh = plsc.VectorSubcoreMesh(core_axis_name="c", subcore_axis_name="s")

    @functools.partial(
        pl.kernel, mesh=mesh,
        out_type=[jax.ShapeDtypeStruct((n_rows, d), jnp.float32),
                  jax.ShapeDtypeStruct((n_rows, d), jnp.float32)],
        scratch_types=[pltpu.VMEM((chunk,), jnp.int32),
                       pltpu.VMEM((chunk, d), jnp.float32),
                       pltpu.SemaphoreType.DMA],
    )
    def k(ta_hbm, tq_hbm, ia_hbm, iq_hbm, oa, oq, idx_v, rows_v, sem):
        wid = jax.lax.axis_index("s") * nc + jax.lax.axis_index("c")
        for t_hbm, i_hbm, o_hbm in ((ta_hbm, ia_hbm, oa),
                                    (tq_hbm, iq_hbm, oq)):
            for j in range(n_chunks):
                base = wid * rows_per_w + j * chunk
                pltpu.sync_copy(i_hbm.at[pl.ds(base, chunk)], idx_v)
                pltpu.async_copy(t_hbm.at[idx_v], rows_v, sem).wait()
                pltpu.sync_copy(rows_v, o_hbm.at[pl.ds(base, chunk)])

    return k(ta, tq, ia, iq)

POS_MARGIN = 0.1
NEG_MARGIN = 1.4
NEG_KERNEL = 9.0
MASK_TH = 0.5


def _main_kernel(t_a_ref, t_q_ref, pa_ref, pq_ref, corr_ref, valid_ref,
                 idxa_ref, idxq_ref, pos_ref, nega_ref, negq_ref,
                 *, R, D, FH, FW):
    HW = FH * FW
    b = pl.program_id(0)
    t = pl.program_id(1)

    @pl.when(jnp.logical_and(b == 0, t == 0))
    def _():
        pos_ref[...] = jnp.zeros_like(pos_ref)
        nega_ref[...] = jnp.zeros_like(nega_ref)
        negq_ref[...] = jnp.zeros_like(negq_ref)

    vm = (valid_ref[0, 0, 0] == 1).astype(jnp.float32)
    corr = corr_ref[0, 0]
    gt = jnp.clip(corr // 8, 0, FH - 1)

    col = jax.lax.broadcasted_iota(jnp.int32, (R, HW), 1)
    yy16 = (col // FW).astype(jnp.bfloat16)
    xx16 = (col % FW).astype(jnp.bfloat16)

    def one_side(t_ref, p_ref, y, x):
        p16 = (p_ref[0, 0] * vm).astype(jnp.bfloat16)
        mh = jax.lax.dot_general(
            p16 * jnp.bfloat16(-0.5), t_ref[0].astype(jnp.bfloat16),
            (((1,), (1,)), ((), ())),
            precision=jax.lax.Precision.DEFAULT,
            preferred_element_type=jnp.float32)
        fd = 0.5 + mh
        y16 = y.astype(jnp.bfloat16)
        x16 = x.astype(jnp.bfloat16)
        dy = y16 - yy16
        dx = x16 - xx16
        d2 = dy * dy + dx * dx
        fdp = jnp.where(d2 < jnp.bfloat16(81.0), 1e9, fd)
        minv = jnp.min(fdp, axis=1, keepdims=True)
        idx = jnp.argmin(fdp, axis=1).astype(jnp.int32)[:, None]
        dist_neg = minv
        idx_f = idx.astype(jnp.float32)
        ny = jnp.floor(idx_f * jnp.float32(1.0 / FW))
        nx = idx_f - ny * jnp.float32(FW)
        posn = p16.astype(jnp.float32)
        return posn, dist_neg, jnp.concatenate([ny * vm, nx * vm], axis=1)

    posn_a, dneg_a, oidx_a = one_side(t_a_ref, pa_ref, gt[:, 0:1], gt[:, 1:2])
    posn_q, dneg_q, oidx_q = one_side(t_q_ref, pq_ref, gt[:, 2:3], gt[:, 3:4])
    idxa_ref[0, 0] = oidx_a
    idxq_ref[0, 0] = oidx_q

    dist_pos = 0.5 * (1.0 - jnp.sum(posn_a * posn_q, axis=1, keepdims=True))
    pos_ref[...] = pos_ref[...] + vm * jnp.sum(jax.nn.relu(dist_pos - POS_MARGIN))
    nega_ref[...] = nega_ref[...] + vm * jnp.sum(jax.nn.relu(NEG_MARGIN - dneg_a))
    negq_ref[...] = negq_ref[...] + vm * jnp.sum(jax.nn.relu(NEG_MARGIN - dneg_q))


def _mask_kernel(la_ref, lq_ref, ma_ref, mq_ref,
                 pma_ref, pmq_ref, bcea_ref, bceq_ref, ioua_ref, iouq_ref,
                 *, B, FH, FW):
    b = pl.program_id(0)

    @pl.when(b == 0)
    def _():
        bcea_ref[...] = jnp.zeros_like(bcea_ref)
        bceq_ref[...] = jnp.zeros_like(bceq_ref)
        ioua_ref[...] = jnp.zeros_like(ioua_ref)
        iouq_ref[...] = jnp.zeros_like(iouq_ref)

    last = b == B - 1

    def one(l_ref, m_ref, pm_ref, bce_ref, iou_ref):
        x = l_ref[0]
        z = m_ref[0].astype(jnp.float32)
        bce = jnp.sum(jax.nn.relu(x) - x * z + jnp.log1p(jnp.exp(-jnp.abs(x))))
        tot = bce_ref[...] + bce
        bce_ref[...] = jnp.where(last, tot / (B * FH * FW), tot)
        pred = (jax.nn.sigmoid(x) > MASK_TH).astype(jnp.int32)
        pm_ref[0] = pred
        pf = pred.astype(jnp.float32)
        inter = jnp.sum(z * pf)
        union = jnp.sum(jnp.clip(z + pf, 0.0, 1.0))
        itot = iou_ref[...] + inter / (union + 1e-6)
        iou_ref[...] = jnp.where(last, itot / B, itot)

    one(la_ref, ma_ref, pma_ref, bcea_ref, ioua_ref)
    one(lq_ref, mq_ref, pmq_ref, bceq_ref, iouq_ref)


def kernel(featmap_a, featmap_q, mask_a_logits, mask_q_logits, corrs, valid,
           anchor_rgb, anchor_mask, query_mask):
    B, D, FH, FW = featmap_a.shape
    HW = FH * FW
    N = corrs.shape[1]
    R = 1024
    NT = N // R

    fa = featmap_a.reshape(B, D, HW)
    fq = featmap_q.reshape(B, D, HW)
    corr4 = corrs.reshape(B, NT, R, 4)
    valid3 = valid.reshape(B, 1, 1)

    prep = pl.pallas_call(
        functools.partial(_prep_kernel, D=D, FH=FH, FW=FW),
        grid=(B,),
        in_specs=[
            pl.BlockSpec((1, D, HW), lambda b: (b, 0, 0)),
            pl.BlockSpec((1, D, HW), lambda b: (b, 0, 0)),
            pl.BlockSpec((1, N, 4), lambda b: (b, 0, 0)),
        ],
        out_specs=[
            pl.BlockSpec((1, HW, D), lambda b: (b, 0, 0)),
            pl.BlockSpec((1, HW, D), lambda b: (b, 0, 0)),
            pl.BlockSpec((1, N, 1), lambda b: (b, 0, 0)),
            pl.BlockSpec((1, N, 1), lambda b: (b, 0, 0)),
        ],
        out_shape=[
            jax.ShapeDtypeStruct((B, HW, D), jnp.float32),
            jax.ShapeDtypeStruct((B, HW, D), jnp.float32),
            jax.ShapeDtypeStruct((B, N, 1), jnp.int32),
            jax.ShapeDtypeStruct((B, N, 1), jnp.int32),
        ],
    )
    ta, tq, ia, iq = prep(fa, fq, corrs)

    oa, oq = _sc_gather_call(ta.reshape(B * HW, D), tq.reshape(B * HW, D),
                             ia.reshape(B * N), iq.reshape(B * N), B * N, D)
    pa = oa.reshape(B, NT, R, D)
    pq = oq.reshape(B, NT, R, D)

    main = pl.pallas_call(
        functools.partial(_main_kernel, R=R, D=D, FH=FH, FW=FW),
        grid=(B, NT),
        in_specs=[
            pl.BlockSpec((1, HW, D), lambda b, t: (b, 0, 0)),
            pl.BlockSpec((1, HW, D), lambda b, t: (b, 0, 0)),
            pl.BlockSpec((1, 1, R, D), lambda b, t: (b, t, 0, 0)),
            pl.BlockSpec((1, 1, R, D), lambda b, t: (b, t, 0, 0)),
            pl.BlockSpec((1, 1, R, 4), lambda b, t: (b, t, 0, 0)),
            pl.BlockSpec((1, 1, 1), lambda b, t: (b, 0, 0)),
        ],
        out_specs=[
            pl.BlockSpec((1, 1, R, 2), lambda b, t: (b, t, 0, 0)),
            pl.BlockSpec((1, 1, R, 2), lambda b, t: (b, t, 0, 0)),
            pl.BlockSpec((1, 1), lambda b, t: (0, 0)),
            pl.BlockSpec((1, 1), lambda b, t: (0, 0)),
            pl.BlockSpec((1, 1), lambda b, t: (0, 0)),
        ],
        out_shape=[
            jax.ShapeDtypeStruct((B, NT, R, 2), jnp.float32),
            jax.ShapeDtypeStruct((B, NT, R, 2), jnp.float32),
            jax.ShapeDtypeStruct((1, 1), jnp.float32),
            jax.ShapeDtypeStruct((1, 1), jnp.float32),
            jax.ShapeDtypeStruct((1, 1), jnp.float32),
        ],
    )
    idxa, idxq, pos_s, nega_s, negq_s = main(ta, tq, pa, pq, corr4, valid3)

    ma = anchor_mask[:, ::8, ::8]
    mq = query_mask[:, ::8, ::8]
    la = mask_a_logits.reshape(B, FH, FW)
    lq = mask_q_logits.reshape(B, FH, FW)
    mask_out = pl.pallas_call(
        functools.partial(_mask_kernel, B=B, FH=FH, FW=FW),
        grid=(B,),
        in_specs=[
            pl.BlockSpec((1, FH, FW), lambda b: (b, 0, 0)),
            pl.BlockSpec((1, FH, FW), lambda b: (b, 0, 0)),
            pl.BlockSpec((1, FH, FW), lambda b: (b, 0, 0)),
            pl.BlockSpec((1, FH, FW), lambda b: (b, 0, 0)),
        ],
        out_specs=[
            pl.BlockSpec((1, FH, FW), lambda b: (b, 0, 0)),
            pl.BlockSpec((1, FH, FW), lambda b: (b, 0, 0)),
            pl.BlockSpec((1, 1), lambda b: (0, 0)),
            pl.BlockSpec((1, 1), lambda b: (0, 0)),
            pl.BlockSpec((1, 1), lambda b: (0, 0)),
            pl.BlockSpec((1, 1), lambda b: (0, 0)),
        ],
        out_shape=[
            jax.ShapeDtypeStruct((B, FH, FW), jnp.int32),
            jax.ShapeDtypeStruct((B, FH, FW), jnp.int32),
            jax.ShapeDtypeStruct((1, 1), jnp.float32),
            jax.ShapeDtypeStruct((1, 1), jnp.float32),
            jax.ShapeDtypeStruct((1, 1), jnp.float32),
            jax.ShapeDtypeStruct((1, 1), jnp.float32),
        ],
    )
    pm_a, pm_q, bce_a, bce_q, iou_a, iou_q = mask_out(la, lq, ma, mq)

    vmask = (valid == 1).astype(jnp.float32)
    cnt = jnp.sum(vmask)
    denom = jnp.maximum(cnt, 1.0)
    pos_loss = jnp.where(cnt > 0, (pos_s[0, 0] / N) / denom, 0.0)
    neg_loss_a = jnp.where(cnt > 0, (nega_s[0, 0] / N) / denom, 0.0)
    neg_loss_q = jnp.where(cnt > 0, (negq_s[0, 0] / N) / denom, 0.0)
    losses = jnp.stack([0.5 * (bce_a[0, 0] + bce_q[0, 0]), pos_loss,
                        0.5 * (neg_loss_a + neg_loss_q)])
    return (losses,
            idxa.reshape(B, N, 2), idxq.reshape(B, N, 2),
            pm_a, pm_q,
            iou_a[0, 0], iou_q[0, 0])
